# Initial kernel scaffold; baseline (speedup 1.0000x reference)
import jax
import jax.numpy as jnp
from jax.experimental import pallas as pl


def kernel(x, pe_table):
    raise NotImplementedError("write your pallas kernel here")


# speedup vs baseline: 1.6161x; 1.6161x over previous
import jax
import jax.numpy as jnp
from jax.experimental import pallas as pl


def _add_body(x_ref, pe_ref, o_ref):
    o_ref[...] = x_ref[...] + pe_ref[...]


def kernel(x, pe_table):
    B, S, D = x.shape
    xf = x.reshape(B * S, D)
    BLK = 512
    n = (B * S) // BLK
    per = S // BLK
    out = pl.pallas_call(
        _add_body,
        out_shape=jax.ShapeDtypeStruct((B * S, D), x.dtype),
        grid=(n,),
        in_specs=[
            pl.BlockSpec((BLK, D), lambda i: (i, 0)),
            pl.BlockSpec((BLK, D), lambda i: (i % per, 0)),
        ],
        out_specs=pl.BlockSpec((BLK, D), lambda i: (i, 0)),
    )(xf, pe_table)
    return out.reshape(B, S, D)


# speedup vs baseline: 2.1673x; 1.3411x over previous
import jax
import jax.numpy as jnp
from jax.experimental import pallas as pl


def _add_body(x_ref, pe_ref, o_ref):
    o_ref[...] = x_ref[...] + pe_ref[...][None, :, :]


def kernel(x, pe_table):
    B, S, D = x.shape
    BLK = 512
    n = S // BLK
    return pl.pallas_call(
        _add_body,
        out_shape=jax.ShapeDtypeStruct((B, S, D), x.dtype),
        grid=(n,),
        in_specs=[
            pl.BlockSpec((B, BLK, D), lambda i: (0, i, 0)),
            pl.BlockSpec((BLK, D), lambda i: (i, 0)),
        ],
        out_specs=pl.BlockSpec((B, BLK, D), lambda i: (0, i, 0)),
    )(x, pe_table)
